# baseline (speedup 1.0000x reference)
import functools

import jax
import jax.numpy as jnp
from jax.experimental import pallas as pl
from jax.experimental.pallas import tpu as pltpu

_TOPK = 100
_CHUNK = 1024


def _topk_body(logits_ref, pbu_ref, scale_ref,
               scores_ref, labels_ref, tb_ref, bxu_ref,
               prob_ref, m_ref, *, n, c, nchunks):
    prob_ref[...] = jax.nn.sigmoid(logits_ref[0])
    mflat = jnp.max(prob_ref[...], axis=(1, 2), keepdims=False)
    pad = jnp.full((1024 - nchunks,), -1.0, jnp.float32)
    m_ref[...] = jnp.concatenate([mflat, pad], axis=0).reshape(8, 128)

    mj_iota = jax.lax.broadcasted_iota(jnp.int32, (8, 128), 0) * 128 + \
        jax.lax.broadcasted_iota(jnp.int32, (8, 128), 1)
    k_iota = jax.lax.broadcasted_iota(jnp.int32, (1, 128), 1)
    r_iota = jax.lax.broadcasted_iota(jnp.int32, (1, 8, 128), 1)
    c_iota = jax.lax.broadcasted_iota(jnp.int32, (1, 8, 128), 2)
    big = jnp.int32(2 ** 30)

    def body(k, carry):
        sc, lb, tbv = carry
        mv = m_ref[...]
        gmax = jnp.max(mv)
        j = jnp.min(jnp.where(mv == gmax, mj_iota, big))
        chunk = prob_ref[pl.ds(j, 1)]
        flat = j * _CHUNK + r_iota * 128 + c_iota
        p = jnp.min(jnp.where(chunk == gmax, flat, big))
        new_chunk = jnp.where(flat == p, -1.0, chunk)
        prob_ref[pl.ds(j, 1)] = new_chunk
        m_ref[...] = jnp.where(mj_iota == j, jnp.max(new_chunk), mv)
        hit = k_iota == k
        return (jnp.where(hit, gmax, sc),
                jnp.where(hit, p % c, lb),
                jnp.where(hit, p // c, tbv))

    zf = jnp.zeros((1, 128), jnp.float32)
    zi = jnp.zeros((1, 128), jnp.int32)
    sc, lb, tbv = jax.lax.fori_loop(0, _TOPK, body, (zf, zi, zi))
    scores_ref[...] = sc[None]
    labels_ref[...] = lb[None]
    tb_ref[...] = tbv[None]

    b = pbu_ref[0]
    cx, cy, w, h = b[0:1, :], b[1:2, :], b[2:3, :], b[3:4, :]
    xyxy = jnp.concatenate(
        [cx - 0.5 * w, cy - 0.5 * h, cx + 0.5 * w, cy + 0.5 * h], axis=0)
    bxu_ref[0] = xyxy * scale_ref[0]


def _cluster_body(tb_ref, mask_hbm, bxu_ref, pb_hbm, scale_ref,
                  clustered_ref, boxes_ref, mrows_ref, pbrows_ref,
                  sem_m, sem_b, *, n):
    bb = pl.program_id(0)

    def issue(k, _):
        row = bb * n + tb_ref[bb, k]
        pltpu.make_async_copy(
            mask_hbm.at[row], mrows_ref.at[k], sem_m).start()
        pltpu.make_async_copy(
            pb_hbm.at[row], pbrows_ref.at[k], sem_b).start()
        return 0

    jax.lax.fori_loop(0, _TOPK, issue, 0, unroll=4)

    bx = bxu_ref[0]

    def drain(k, _):
        pltpu.make_async_copy(
            mask_hbm.at[0], mrows_ref.at[k], sem_m).wait()
        pltpu.make_async_copy(
            pb_hbm.at[0], pbrows_ref.at[k], sem_b).wait()
        return 0

    jax.lax.fori_loop(0, _TOPK, drain, 0, unroll=4)

    def consume(k, _):
        mrow = mrows_ref[k] != 0
        clustered_ref[0, pl.ds(k, 1)] = jnp.where(mrow, bx, 0.0)[None]
        return 0

    jax.lax.fori_loop(0, _TOPK, consume, 0, unroll=4)

    r = pbrows_ref[:, 0, :]
    cx, cy, w, h = r[:, 0:1], r[:, 1:2], r[:, 2:3], r[:, 3:4]
    xyxy = jnp.concatenate(
        [cx - 0.5 * w, cy - 0.5 * h, cx + 0.5 * w, cy + 0.5 * h], axis=1)
    boxes_ref[...] = (xyxy * scale_ref[...].reshape(1, 4))[None]


@jax.jit
def kernel(pred_logits, pred_boxes, pred_logits_unaggregated,
           pred_boxes_unaggregated, aggregation_mask, target_sizes):
    del pred_logits_unaggregated
    b, n, c = pred_logits.shape
    nc = n * c
    nchunks = (nc + _CHUNK - 1) // _CHUNK
    pad = nchunks * _CHUNK - nc

    img_h = target_sizes[:, 0].astype(jnp.float32)
    img_w = target_sizes[:, 1].astype(jnp.float32)
    scale = jnp.stack([img_w, img_h, img_w, img_h], axis=1)[:, None, :]

    lflat = jnp.pad(pred_logits.reshape(b, nc), ((0, 0), (0, pad)),
                    constant_values=-1e9).reshape(b, nchunks, 8, 128)

    pbuT = jnp.swapaxes(pred_boxes_unaggregated, 1, 2)
    scaleT = scale.reshape(b, 4, 1)

    k1 = pl.pallas_call(
        functools.partial(_topk_body, n=n, c=c, nchunks=nchunks),
        grid=(b,),
        in_specs=[
            pl.BlockSpec((1, nchunks, 8, 128), lambda i: (i, 0, 0, 0)),
            pl.BlockSpec((1, 4, n), lambda i: (i, 0, 0)),
            pl.BlockSpec((1, 4, 1), lambda i: (i, 0, 0)),
        ],
        out_specs=[
            pl.BlockSpec((1, 1, 128), lambda i: (i, 0, 0)),
            pl.BlockSpec((1, 1, 128), lambda i: (i, 0, 0)),
            pl.BlockSpec((1, 1, 128), lambda i: (i, 0, 0)),
            pl.BlockSpec((1, 4, n), lambda i: (i, 0, 0)),
        ],
        out_shape=[
            jax.ShapeDtypeStruct((b, 1, 128), jnp.float32),
            jax.ShapeDtypeStruct((b, 1, 128), jnp.int32),
            jax.ShapeDtypeStruct((b, 1, 128), jnp.int32),
            jax.ShapeDtypeStruct((b, 4, n), jnp.float32),
        ],
        scratch_shapes=[
            pltpu.VMEM((nchunks, 8, 128), jnp.float32),
            pltpu.VMEM((8, 128), jnp.float32),
        ],
    )(lflat, pbuT, scaleT)
    scores3, labels3, tb3, bxu = k1

    tb128 = tb3[:, 0, :]

    clusteredT, boxes = pl.pallas_call(
        functools.partial(_cluster_body, n=n),
        grid=(b,),
        in_specs=[
            pl.BlockSpec(memory_space=pltpu.MemorySpace.SMEM),
            pl.BlockSpec(memory_space=pltpu.MemorySpace.HBM),
            pl.BlockSpec((1, 4, n), lambda i: (i, 0, 0)),
            pl.BlockSpec(memory_space=pltpu.MemorySpace.HBM),
            pl.BlockSpec((1, 1, 4), lambda i: (i, 0, 0)),
        ],
        out_specs=[
            pl.BlockSpec((1, _TOPK, 4, n), lambda i: (i, 0, 0, 0)),
            pl.BlockSpec((1, _TOPK, 4), lambda i: (i, 0, 0)),
        ],
        out_shape=[
            jax.ShapeDtypeStruct((b, _TOPK, 4, n), jnp.float32),
            jax.ShapeDtypeStruct((b, _TOPK, 4), jnp.float32),
        ],
        scratch_shapes=[
            pltpu.VMEM((_TOPK, 1, n), jnp.uint8),
            pltpu.VMEM((_TOPK, 1, 4), jnp.float32),
            pltpu.SemaphoreType.DMA,
            pltpu.SemaphoreType.DMA,
        ],
    )(tb128, aggregation_mask.view(jnp.uint8).reshape(b * n, 1, n), bxu,
      pred_boxes.reshape(b * n, 1, 4), scale)

    return (scores3[:, 0, :_TOPK], labels3[:, 0, :_TOPK],
            boxes, jnp.swapaxes(clusteredT, 2, 3))


# speedup vs baseline: 1.5277x; 1.5277x over previous
import functools

import jax
import jax.numpy as jnp
from jax.experimental import pallas as pl
from jax.experimental.pallas import tpu as pltpu

_TOPK = 100
_CHUNK = 1024


def _topk_body(logits_ref, pbu_ref, scale_ref,
               scores_ref, labels_ref, tb_ref, bxu_ref,
               prob_ref, m_ref, *, n, c, nchunks):
    prob_ref[...] = jax.nn.sigmoid(logits_ref[0])
    mflat = jnp.max(prob_ref[...], axis=(1, 2), keepdims=False)
    pad = jnp.full((1024 - nchunks,), -1.0, jnp.float32)
    m_ref[...] = jnp.concatenate([mflat, pad], axis=0).reshape(8, 128)

    mj_iota = jax.lax.broadcasted_iota(jnp.int32, (8, 128), 0) * 128 + \
        jax.lax.broadcasted_iota(jnp.int32, (8, 128), 1)
    k_iota = jax.lax.broadcasted_iota(jnp.int32, (1, 128), 1)
    r_iota = jax.lax.broadcasted_iota(jnp.int32, (1, 8, 128), 1)
    c_iota = jax.lax.broadcasted_iota(jnp.int32, (1, 8, 128), 2)
    big = jnp.int32(2 ** 30)

    def body(k, carry):
        sc, lb, tbv = carry
        mv = m_ref[...]
        gmax = jnp.max(mv)
        j = jnp.min(jnp.where(mv == gmax, mj_iota, big))
        chunk = prob_ref[pl.ds(j, 1)]
        flat = j * _CHUNK + r_iota * 128 + c_iota
        p = jnp.min(jnp.where(chunk == gmax, flat, big))
        new_chunk = jnp.where(flat == p, -1.0, chunk)
        prob_ref[pl.ds(j, 1)] = new_chunk
        m_ref[...] = jnp.where(mj_iota == j, jnp.max(new_chunk), mv)
        hit = k_iota == k
        return (jnp.where(hit, gmax, sc),
                jnp.where(hit, p % c, lb),
                jnp.where(hit, p // c, tbv))

    zf = jnp.zeros((1, 128), jnp.float32)
    zi = jnp.zeros((1, 128), jnp.int32)
    sc, lb, tbv = jax.lax.fori_loop(0, _TOPK, body, (zf, zi, zi))
    scores_ref[...] = sc[None]
    labels_ref[...] = lb[None]
    tb_ref[...] = tbv[None]

    b = pbu_ref[0]
    cx, cy, w, h = b[0:1, :], b[1:2, :], b[2:3, :], b[3:4, :]
    xyxy = jnp.concatenate(
        [cx - 0.5 * w, cy - 0.5 * h, cx + 0.5 * w, cy + 0.5 * h], axis=0)
    bxu_ref[0] = xyxy * scale_ref[0]


def _cluster_body(tb_ref, m0, m1, m2, m3, bxu_ref, p0, p1, p2, p3, scale_ref,
                  clustered_ref, boxes_ref):
    bx = bxu_ref[0]
    for s, m in enumerate((m0, m1, m2, m3)):
        mrow = m[0, 0]
        clustered_ref[0, s] = jnp.where(mrow, bx, 0.0)

    r = jnp.concatenate(
        [p[...].reshape(1, 4) for p in (p0, p1, p2, p3)], axis=0)
    cx, cy, w, h = r[:, 0:1], r[:, 1:2], r[:, 2:3], r[:, 3:4]
    xyxy = jnp.concatenate(
        [cx - 0.5 * w, cy - 0.5 * h, cx + 0.5 * w, cy + 0.5 * h], axis=1)
    boxes_ref[...] = (xyxy * scale_ref[...].reshape(1, 4))[None, None]


@jax.jit
def kernel(pred_logits, pred_boxes, pred_logits_unaggregated,
           pred_boxes_unaggregated, aggregation_mask, target_sizes):
    del pred_logits_unaggregated
    b, n, c = pred_logits.shape
    nc = n * c
    nchunks = (nc + _CHUNK - 1) // _CHUNK
    pad = nchunks * _CHUNK - nc

    img_h = target_sizes[:, 0].astype(jnp.float32)
    img_w = target_sizes[:, 1].astype(jnp.float32)
    scale = jnp.stack([img_w, img_h, img_w, img_h], axis=1)[:, None, :]

    lflat = jnp.pad(pred_logits.reshape(b, nc), ((0, 0), (0, pad)),
                    constant_values=-1e9).reshape(b, nchunks, 8, 128)

    pbuT = jnp.swapaxes(pred_boxes_unaggregated, 1, 2)
    scaleT = scale.reshape(b, 4, 1)

    k1 = pl.pallas_call(
        functools.partial(_topk_body, n=n, c=c, nchunks=nchunks),
        grid=(b,),
        in_specs=[
            pl.BlockSpec((1, nchunks, 8, 128), lambda i: (i, 0, 0, 0)),
            pl.BlockSpec((1, 4, n), lambda i: (i, 0, 0)),
            pl.BlockSpec((1, 4, 1), lambda i: (i, 0, 0)),
        ],
        out_specs=[
            pl.BlockSpec((1, 1, 128), lambda i: (i, 0, 0)),
            pl.BlockSpec((1, 1, 128), lambda i: (i, 0, 0)),
            pl.BlockSpec((1, 1, 128), lambda i: (i, 0, 0)),
            pl.BlockSpec((1, 4, n), lambda i: (i, 0, 0)),
        ],
        out_shape=[
            jax.ShapeDtypeStruct((b, 1, 128), jnp.float32),
            jax.ShapeDtypeStruct((b, 1, 128), jnp.int32),
            jax.ShapeDtypeStruct((b, 1, 128), jnp.int32),
            jax.ShapeDtypeStruct((b, 4, n), jnp.float32),
        ],
        scratch_shapes=[
            pltpu.VMEM((nchunks, 8, 128), jnp.float32),
            pltpu.VMEM((8, 128), jnp.float32),
        ],
    )(lflat, pbuT, scaleT)
    scores3, labels3, tb3, bxu = k1
    tb = tb3[:, 0, :_TOPK]

    mask4 = aggregation_mask.reshape(b, n, 1, n)
    pb4 = pred_boxes.reshape(b, n, 1, 4)

    def mspec(s):
        return pl.BlockSpec(
            (1, 1, 1, n), lambda i, k, tbr, s=s: (i, tbr[i, 4 * k + s], 0, 0))

    def pspec(s):
        return pl.BlockSpec(
            (1, 1, 1, 4), lambda i, k, tbr, s=s: (i, tbr[i, 4 * k + s], 0, 0))

    grid_spec = pltpu.PrefetchScalarGridSpec(
        num_scalar_prefetch=1,
        grid=(b, _TOPK // 4),
        in_specs=[
            mspec(0), mspec(1), mspec(2), mspec(3),
            pl.BlockSpec((1, 4, n), lambda i, k, tbr: (i, 0, 0)),
            pspec(0), pspec(1), pspec(2), pspec(3),
            pl.BlockSpec((1, 1, 4), lambda i, k, tbr: (i, 0, 0)),
        ],
        out_specs=[
            pl.BlockSpec((1, 4, 4, n), lambda i, k, tbr: (i, k, 0, 0)),
            pl.BlockSpec((1, 1, 4, 4), lambda i, k, tbr: (i, k, 0, 0)),
        ],
    )
    clusteredT, boxes = pl.pallas_call(
        _cluster_body,
        grid_spec=grid_spec,
        out_shape=[
            jax.ShapeDtypeStruct((b, _TOPK, 4, n), jnp.float32),
            jax.ShapeDtypeStruct((b, _TOPK // 4, 4, 4), jnp.float32),
        ],
    )(tb, mask4, mask4, mask4, mask4, bxu, pb4, pb4, pb4, pb4, scale)

    return (scores3[:, 0, :_TOPK], labels3[:, 0, :_TOPK],
            boxes.reshape(b, _TOPK, 4), jnp.swapaxes(clusteredT, 2, 3))


# speedup vs baseline: 1.5683x; 1.0266x over previous
import functools

import jax
import jax.numpy as jnp
from jax.experimental import pallas as pl
from jax.experimental.pallas import tpu as pltpu

_TOPK = 100
_CHUNK = 1024


def _topk_body(logits_ref, pbu_ref, scale_ref,
               scores_ref, labels_ref, tb_ref, bxu_ref,
               prob_ref, m_ref, *, n, c, nchunks):
    prob_ref[...] = jax.nn.sigmoid(logits_ref[0])
    mflat = jnp.max(prob_ref[...], axis=(1, 2), keepdims=False)
    pad = jnp.full((1024 - nchunks,), -1.0, jnp.float32)
    m_ref[...] = jnp.concatenate([mflat, pad], axis=0).reshape(8, 128)

    mj_iota = jax.lax.broadcasted_iota(jnp.int32, (8, 128), 0) * 128 + \
        jax.lax.broadcasted_iota(jnp.int32, (8, 128), 1)
    k_iota = jax.lax.broadcasted_iota(jnp.int32, (1, 128), 1)
    r_iota = jax.lax.broadcasted_iota(jnp.int32, (1, 8, 128), 1)
    c_iota = jax.lax.broadcasted_iota(jnp.int32, (1, 8, 128), 2)
    big = jnp.int32(2 ** 30)

    def body(k, carry):
        sc, lb, tbv = carry
        mv = m_ref[...]
        gmax = jnp.max(mv)
        j = jnp.min(jnp.where(mv == gmax, mj_iota, big))
        chunk = prob_ref[pl.ds(j, 1)]
        flat = j * _CHUNK + r_iota * 128 + c_iota
        p = jnp.min(jnp.where(chunk == gmax, flat, big))
        new_chunk = jnp.where(flat == p, -1.0, chunk)
        prob_ref[pl.ds(j, 1)] = new_chunk
        m_ref[...] = jnp.where(mj_iota == j, jnp.max(new_chunk), mv)
        hit = k_iota == k
        return (jnp.where(hit, gmax, sc),
                jnp.where(hit, p % c, lb),
                jnp.where(hit, p // c, tbv))

    zf = jnp.zeros((1, 128), jnp.float32)
    zi = jnp.zeros((1, 128), jnp.int32)
    sc, lb, tbv = jax.lax.fori_loop(0, _TOPK, body, (zf, zi, zi))
    scores_ref[...] = sc[None]
    labels_ref[...] = lb[None]
    tb_ref[...] = tbv[None]

    b = pbu_ref[0]
    cx, cy, w, h = b[0:1, :], b[1:2, :], b[2:3, :], b[3:4, :]
    xyxy = jnp.concatenate(
        [cx - 0.5 * w, cy - 0.5 * h, cx + 0.5 * w, cy + 0.5 * h], axis=0)
    bxu_ref[0] = xyxy * scale_ref[0]


_KPG = 10


def _cluster_body(tb_ref, *refs):
    mrefs = refs[:_KPG]
    bxu_ref = refs[_KPG]
    prefs = refs[_KPG + 1:2 * _KPG + 1]
    scale_ref = refs[2 * _KPG + 1]
    clustered_ref, boxes_ref = refs[2 * _KPG + 2], refs[2 * _KPG + 3]

    bx = bxu_ref[0]
    for s, m in enumerate(mrefs):
        mrow = m[0, 0]
        clustered_ref[0, s] = jnp.where(mrow, bx, 0.0)

    r = jnp.concatenate(
        [p[...].reshape(1, 4) for p in prefs], axis=0)
    cx, cy, w, h = r[:, 0:1], r[:, 1:2], r[:, 2:3], r[:, 3:4]
    xyxy = jnp.concatenate(
        [cx - 0.5 * w, cy - 0.5 * h, cx + 0.5 * w, cy + 0.5 * h], axis=1)
    boxes_ref[...] = (xyxy * scale_ref[...].reshape(1, 4))[None, None]


@jax.jit
def kernel(pred_logits, pred_boxes, pred_logits_unaggregated,
           pred_boxes_unaggregated, aggregation_mask, target_sizes):
    del pred_logits_unaggregated
    b, n, c = pred_logits.shape
    nc = n * c
    nchunks = (nc + _CHUNK - 1) // _CHUNK
    pad = nchunks * _CHUNK - nc

    img_h = target_sizes[:, 0].astype(jnp.float32)
    img_w = target_sizes[:, 1].astype(jnp.float32)
    scale = jnp.stack([img_w, img_h, img_w, img_h], axis=1)[:, None, :]

    lflat = jnp.pad(pred_logits.reshape(b, nc), ((0, 0), (0, pad)),
                    constant_values=-1e9).reshape(b, nchunks, 8, 128)

    pbuT = jnp.swapaxes(pred_boxes_unaggregated, 1, 2)
    scaleT = scale.reshape(b, 4, 1)

    k1 = pl.pallas_call(
        functools.partial(_topk_body, n=n, c=c, nchunks=nchunks),
        grid=(b,),
        in_specs=[
            pl.BlockSpec((1, nchunks, 8, 128), lambda i: (i, 0, 0, 0)),
            pl.BlockSpec((1, 4, n), lambda i: (i, 0, 0)),
            pl.BlockSpec((1, 4, 1), lambda i: (i, 0, 0)),
        ],
        out_specs=[
            pl.BlockSpec((1, 1, 128), lambda i: (i, 0, 0)),
            pl.BlockSpec((1, 1, 128), lambda i: (i, 0, 0)),
            pl.BlockSpec((1, 1, 128), lambda i: (i, 0, 0)),
            pl.BlockSpec((1, 4, n), lambda i: (i, 0, 0)),
        ],
        out_shape=[
            jax.ShapeDtypeStruct((b, 1, 128), jnp.float32),
            jax.ShapeDtypeStruct((b, 1, 128), jnp.int32),
            jax.ShapeDtypeStruct((b, 1, 128), jnp.int32),
            jax.ShapeDtypeStruct((b, 4, n), jnp.float32),
        ],
        scratch_shapes=[
            pltpu.VMEM((nchunks, 8, 128), jnp.float32),
            pltpu.VMEM((8, 128), jnp.float32),
        ],
    )(lflat, pbuT, scaleT)
    scores3, labels3, tb3, bxu = k1
    tb = tb3[:, 0, :_TOPK]

    mask4 = aggregation_mask.reshape(b, n, 1, n)
    pb4 = pred_boxes.reshape(b, n, 1, 4)

    def mspec(s):
        return pl.BlockSpec(
            (1, 1, 1, n),
            lambda i, k, tbr, s=s: (i, tbr[i, _KPG * k + s], 0, 0))

    def pspec(s):
        return pl.BlockSpec(
            (1, 1, 1, 4),
            lambda i, k, tbr, s=s: (i, tbr[i, _KPG * k + s], 0, 0))

    grid_spec = pltpu.PrefetchScalarGridSpec(
        num_scalar_prefetch=1,
        grid=(b, _TOPK // _KPG),
        in_specs=(
            [mspec(s) for s in range(_KPG)]
            + [pl.BlockSpec((1, 4, n), lambda i, k, tbr: (i, 0, 0))]
            + [pspec(s) for s in range(_KPG)]
            + [pl.BlockSpec((1, 1, 4), lambda i, k, tbr: (i, 0, 0))]
        ),
        out_specs=[
            pl.BlockSpec((1, _KPG, 4, n), lambda i, k, tbr: (i, k, 0, 0)),
            pl.BlockSpec((1, 1, _KPG, 4), lambda i, k, tbr: (i, k, 0, 0)),
        ],
    )
    clusteredT, boxes = pl.pallas_call(
        _cluster_body,
        grid_spec=grid_spec,
        out_shape=[
            jax.ShapeDtypeStruct((b, _TOPK, 4, n), jnp.float32),
            jax.ShapeDtypeStruct((b, _TOPK // _KPG, _KPG, 4), jnp.float32),
        ],
    )(tb, *([mask4] * _KPG), bxu, *([pb4] * _KPG), scale)

    return (scores3[:, 0, :_TOPK], labels3[:, 0, :_TOPK],
            boxes.reshape(b, _TOPK, 4), jnp.swapaxes(clusteredT, 2, 3))
